# baseline (speedup 1.0000x reference)
import functools

import jax
import jax.numpy as jnp
from jax import lax
from jax.experimental import pallas as pl
from jax.experimental.pallas import tpu as pltpu
from jax.experimental.pallas import tpu_sc as plsc

NUM_USER = 1000000
NUM_ITEM = 100000
NUM_TIME = 200
D = 32
BATCH = 16384

NC = 2
NS = 16
LANES = 16
NW = NC * NS
BPW = BATCH // NW
NV = BPW // LANES
GCH = 128
NCH = (BPW * D) // GCH
LAG = 8


def _body(u_hbm, v_hbm, t_hbm, i_hbm, j_hbm, k_hbm, out_hbm,
          iv, jv, kv, idxu, idxv, ubuf, vbuf, tbuf, outv, sem):
  wid = lax.axis_index("s") * NC + lax.axis_index("c")
  base = wid * BPW

  pltpu.sync_copy(i_hbm.at[pl.ds(base, BPW)], iv)
  pltpu.sync_copy(j_hbm.at[pl.ds(base, BPW)], jv)
  pltpu.sync_copy(k_hbm.at[pl.ds(base, BPW)], kv)
  pltpu.sync_copy(t_hbm, tbuf)

  for d in range(D):
    for nv_i in range(NV):
      flat = d * BPW + nv_i * LANES
      r, c = flat // GCH, flat % GCH
      idxu[r, pl.ds(c, LANES)] = iv[pl.ds(nv_i * LANES, LANES)] + d * NUM_USER
      idxv[r, pl.ds(c, LANES)] = jv[pl.ds(nv_i * LANES, LANES)] + d * NUM_ITEM

  def fire(c):
    return (
        pltpu.async_copy(u_hbm.at[idxu.at[c]],
                         ubuf.at[pl.ds(c * GCH, GCH)], sem),
        pltpu.async_copy(v_hbm.at[idxv.at[c]],
                         vbuf.at[pl.ds(c * GCH, GCH)], sem),
    )

  def wait_for(c):
    pltpu.make_async_copy(u_hbm.at[idxu.at[c]],
                          ubuf.at[pl.ds(c * GCH, GCH)], sem).wait()
    pltpu.make_async_copy(v_hbm.at[idxv.at[c]],
                          vbuf.at[pl.ds(c * GCH, GCH)], sem).wait()

  def gstep(c, carry):
    fire(c)
    @pl.when(c >= LAG)
    def _():
      wait_for(c - LAG)
    return carry

  lax.fori_loop(0, NCH, gstep, 0)

  def dstep(c, carry):
    wait_for(c)
    return carry

  lax.fori_loop(NCH - LAG, NCH, dstep, 0)

  def compute(nv_i, carry):
    nbase = nv_i * LANES
    kvv = kv[pl.ds(nbase, LANES)]
    acc = jnp.zeros((LANES,), jnp.float32)
    for d in range(D):
      uu = ubuf[pl.ds(d * BPW + nbase, LANES)]
      vv = vbuf[pl.ds(d * BPW + nbase, LANES)]
      tt = plsc.load_gather(tbuf, [kvv + d * NUM_TIME])
      acc = acc + uu * vv * tt
    outv[pl.ds(nbase, LANES)] = acc
    return carry

  lax.fori_loop(0, NV, compute, 0)

  pltpu.sync_copy(outv, out_hbm.at[pl.ds(base, BPW)])


@jax.jit
def _run(user_embeddings, item_embeddings, time_embeddings,
         i_input, j_input, k_input):
  mesh = plsc.VectorSubcoreMesh(core_axis_name="c", subcore_axis_name="s")
  f = pl.kernel(
      _body,
      out_type=jax.ShapeDtypeStruct((BATCH,), jnp.float32),
      mesh=mesh,
      compiler_params=pltpu.CompilerParams(
          needs_layout_passes=False, use_tc_tiling_on_sc=False),
      scratch_types=[
          pltpu.VMEM((BPW,), jnp.int32),
          pltpu.VMEM((BPW,), jnp.int32),
          pltpu.VMEM((BPW,), jnp.int32),
          pltpu.VMEM((NCH, GCH), jnp.int32),
          pltpu.VMEM((NCH, GCH), jnp.int32),
          pltpu.VMEM((BPW * D,), jnp.float32),
          pltpu.VMEM((BPW * D,), jnp.float32),
          pltpu.VMEM((NUM_TIME * D,), jnp.float32),
          pltpu.VMEM((BPW,), jnp.float32),
          pltpu.SemaphoreType.DMA,
      ],
  )
  u1 = user_embeddings.T.reshape(-1)
  v1 = item_embeddings.T.reshape(-1)
  t1 = time_embeddings.T.reshape(-1)
  return f(u1, v1, t1, i_input, j_input, k_input)


def kernel(user_embeddings, item_embeddings, time_embeddings,
           i_input, j_input, k_input):
  return _run(user_embeddings, item_embeddings, time_embeddings,
              i_input.astype(jnp.int32), j_input.astype(jnp.int32),
              k_input.astype(jnp.int32))


# speedup vs baseline: 4.8064x; 4.8064x over previous
import functools

import jax
import jax.numpy as jnp
from jax import lax
from jax.experimental import pallas as pl
from jax.experimental.pallas import tpu as pltpu
from jax.experimental.pallas import tpu_sc as plsc

NUM_USER = 1000000
NUM_ITEM = 100000
NUM_TIME = 200
D = 32
BATCH = 16384

NC = 2
NS = 16
LANES = 16
NW = NC * NS
BPW = BATCH // NW
GCH = 128
NG = BPW // GCH


def _body(u_hbm, v_hbm, t_hbm, i_hbm, j_hbm, k_hbm, out_hbm,
          iv, jv, kv, ur, vr, tr, outv, sem):
  wid = lax.axis_index("s") * NC + lax.axis_index("c")
  base = wid * BPW

  for t in range(NG):
    pltpu.sync_copy(i_hbm.at[pl.ds(base + t * GCH, GCH)], iv.at[t])
    pltpu.sync_copy(j_hbm.at[pl.ds(base + t * GCH, GCH)], jv.at[t])
    pltpu.sync_copy(k_hbm.at[pl.ds(base + t * GCH, GCH)], kv.at[t])

  copies = []
  for t in range(NG):
    sl = pl.ds(t * GCH, GCH)
    copies.append(pltpu.async_copy(u_hbm.at[iv.at[t]], ur.at[sl], sem))
    copies.append(pltpu.async_copy(v_hbm.at[jv.at[t]], vr.at[sl], sem))
    copies.append(pltpu.async_copy(t_hbm.at[kv.at[t]], tr.at[sl], sem))
  for c in copies:
    c.wait()

  lane = lax.iota(jnp.int32, LANES)

  def chunk(c, carry):
    base_r = c * LANES
    acc = jnp.zeros((LANES,), jnp.float32)
    for s in range(LANES):
      r = base_r + s
      u0 = ur[r, pl.ds(0, LANES)]
      u1 = ur[r, pl.ds(LANES, LANES)]
      v0 = vr[r, pl.ds(0, LANES)]
      v1 = vr[r, pl.ds(LANES, LANES)]
      t0 = tr[r, pl.ds(0, LANES)]
      t1 = tr[r, pl.ds(LANES, LANES)]
      q = u0 * v0 * t0 + u1 * v1 * t1
      ssum = jnp.full((LANES,), jnp.sum(q), jnp.float32)
      acc = jnp.where(lane == s, ssum, acc)
    outv[pl.ds(base_r, LANES)] = acc
    return carry

  lax.fori_loop(0, BPW // LANES, chunk, 0)

  pltpu.sync_copy(outv, out_hbm.at[pl.ds(base, BPW)])


@functools.partial(jax.jit, static_argnames=())
def _run(user_embeddings, item_embeddings, time_embeddings,
         i_input, j_input, k_input):
  mesh = plsc.VectorSubcoreMesh(core_axis_name="c", subcore_axis_name="s")
  f = pl.kernel(
      _body,
      out_type=jax.ShapeDtypeStruct((BATCH,), jnp.float32),
      mesh=mesh,
      compiler_params=pltpu.CompilerParams(
          needs_layout_passes=False, use_tc_tiling_on_sc=False),
      scratch_types=[
          pltpu.VMEM((NG, GCH), jnp.int32),
          pltpu.VMEM((NG, GCH), jnp.int32),
          pltpu.VMEM((NG, GCH), jnp.int32),
          pltpu.VMEM((BPW, D), jnp.float32),
          pltpu.VMEM((BPW, D), jnp.float32),
          pltpu.VMEM((BPW, D), jnp.float32),
          pltpu.VMEM((BPW,), jnp.float32),
          pltpu.SemaphoreType.DMA,
      ],
  )
  return f(user_embeddings, item_embeddings, time_embeddings,
           i_input, j_input, k_input)


def kernel(user_embeddings, item_embeddings, time_embeddings,
           i_input, j_input, k_input):
  return _run(user_embeddings, item_embeddings, time_embeddings,
              i_input.astype(jnp.int32), j_input.astype(jnp.int32),
              k_input.astype(jnp.int32))
